# baseline (speedup 1.0000x reference)
import jax
import jax.numpy as jnp
from jax import lax
from jax.experimental import pallas as pl
from jax.experimental.pallas import tpu as pltpu
from jax.experimental.pallas import tpu_sc as plsc

NC, NS, LANES = 2, 16, 16
NW = NC * NS
SEQ = 128
D = 64
DP = 128
B = 4096
SEQ_PER_W = B // NW
BLK = D * SEQ
DV = D // LANES
UNROLL = 8


def _emb_kernel(x_hbm, table_hbm, post_hbm, out_hbm, idx_v, post_v, rows_v,
                outt_v, spos, sem0, sem1, semi0, semi1):
    cid = lax.axis_index("c")
    sid = lax.axis_index("s")
    wid = sid * NC + cid
    seq_base = pl.multiple_of(wid * SEQ_PER_W, SEQ_PER_W)

    pltpu.sync_copy(x_hbm.at[pl.ds(seq_base, SEQ_PER_W)], idx_v)
    pltpu.sync_copy(post_hbm, post_v)
    pltpu.sync_copy(post_v, spos.at[sid])

    sems = (sem0, sem1)
    semis = (semi0, semi1)
    lane_off = lax.iota(jnp.int32, LANES) * SEQ

    def fire(s, buf):
        pltpu.async_copy(table_hbm.at[idx_v.at[s]], rows_v.at[buf], sems[buf])

    def drain(buf):
        pltpu.make_async_copy(
            table_hbm.at[pl.ds(0, SEQ)], rows_v.at[buf], sems[buf]
        ).wait()

    def scatter_add(buf):
        def body(i, _):
            for u in range(UNROLL):
                l = i * UNROLL + u
                for c in range(DV):
                    v = rows_v[buf, l, pl.ds(c * LANES, LANES)]
                    idx = lane_off + (c * LANES * SEQ + l)
                    plsc.addupdate_scatter(outt_v.at[buf], [idx], v)
            return 0
        lax.fori_loop(0, SEQ // UNROLL, body, 0)

    def store(s, buf):
        start = pl.multiple_of((seq_base + s) * BLK, BLK)
        pltpu.sync_copy(outt_v.at[buf], out_hbm.at[pl.ds(start, BLK)])

    def fire_init(buf):
        pltpu.async_copy(spos.at[sid], outt_v.at[buf], semis[buf])

    def drain_init(buf):
        pltpu.make_async_copy(spos.at[sid], outt_v.at[buf], semis[buf]).wait()

    fire(0, 0)
    fire_init(0)
    fire_init(1)

    @pl.loop(0, SEQ_PER_W, step=2)
    def _seqs(s0):
        for b in range(2):
            s = s0 + b

            @pl.when(s + 1 < SEQ_PER_W)
            def _():
                fire(s + 1, 1 - b)

            drain_init(b)
            drain(b)
            scatter_add(b)
            store(s, b)

            @pl.when(s + 2 < SEQ_PER_W)
            def _():
                fire_init(b)


def kernel(x, table, pos):
    tablep = jnp.pad(table, ((0, 0), (0, DP - D)))
    post = jnp.swapaxes(pos, 0, 1).reshape(-1)

    mesh = plsc.VectorSubcoreMesh(
        core_axis_name="c", subcore_axis_name="s",
        num_cores=NC, num_subcores=NS,
    )
    f = pl.kernel(
        _emb_kernel,
        out_type=jax.ShapeDtypeStruct((B * BLK,), jnp.float32),
        mesh=mesh,
        scratch_types=[
            pltpu.VMEM((SEQ_PER_W, SEQ), jnp.int32),
            pltpu.VMEM((BLK,), jnp.float32),
            pltpu.VMEM((2, SEQ, DP), jnp.float32),
            pltpu.VMEM((2, BLK), jnp.float32),
            pltpu.VMEM_SHARED((NS, BLK), jnp.float32),
            pltpu.SemaphoreType.DMA,
            pltpu.SemaphoreType.DMA,
            pltpu.SemaphoreType.DMA,
            pltpu.SemaphoreType.DMA,
        ],
        compiler_params=pltpu.CompilerParams(
            use_tc_tiling_on_sc=False, needs_layout_passes=False,
        ),
    )
    out = f(x, tablep, post)
    return jnp.swapaxes(out.reshape(B, D, SEQ), 1, 2)
